# baseline (speedup 1.0000x reference)
import functools

import jax
import jax.numpy as jnp
from jax import lax
from jax.experimental import pallas as pl
from jax.experimental.pallas import tpu as pltpu
from jax.experimental.pallas import tpu_sc as plsc

_NC = 2
_NS = 16
_NW = _NC * _NS


def _build_table(wt, wtail):
    d, vocab = wt.shape
    n_full = vocab // 128
    vocab_pad = (vocab + 127) // 128 * 128
    n_even = (n_full // _NW) * _NW
    k_main = n_even // _NW
    n_extra = n_full - n_even
    tail = vocab - n_full * 128
    blk = 128 * d

    mesh = plsc.VectorSubcoreMesh(
        core_axis_name="c", subcore_axis_name="s")

    @functools.partial(
        pl.kernel,
        mesh=mesh,
        compiler_params=pltpu.CompilerParams(
            use_tc_tiling_on_sc=True, needs_layout_passes=False),
        out_type=jax.ShapeDtypeStruct((vocab_pad * d,), jnp.float32),
        scratch_types=[
            pltpu.VMEM((2, 4, 8, 128), jnp.float32),
            [pltpu.VMEM((blk,), jnp.float32)] * 2,
            pltpu.VMEM((tail * d // 128, 128), jnp.float32),
            pltpu.VMEM((tail * d,), jnp.float32),
            [pltpu.SemaphoreType.DMA] * 2,
            [pltpu.SemaphoreType.DMA] * 2,
        ],
    )
    def body(wt_hbm, wtail_hbm, wf_hbm, tin, touts, tail_v, tail_r,
             isems, osems):
        cid = lax.axis_index("c")
        sid = lax.axis_index("s")
        wid = sid * _NC + cid

        lane32 = lax.iota(jnp.int32, 16) * d

        def fire_in(q, b):
            for a in range(4):
                pltpu.async_copy(
                    wt_hbm.at[pl.ds(8 * a, 8), pl.ds(q * 128, 128)],
                    tin.at[b, a], isems[b])

        def wait_in(b):
            for a in range(4):
                pltpu.make_async_copy(
                    wt_hbm.at[pl.ds(0, 8), pl.ds(0, 128)],
                    tin.at[b, a], isems[b]).wait()

        def transpose(b):
            def col(c, _):
                for m in range(8):
                    v = tin[b, c // 8, c % 8, pl.ds(16 * m, 16)]
                    plsc.store_scatter(
                        touts[b], [lane32 + (16 * m * d + c)], v)
                return ()

            lax.fori_loop(0, d, col, (), unroll=4)

        def fire_out(q, b):
            pltpu.async_copy(
                touts[b], wf_hbm.at[pl.ds(q * blk, blk)], osems[b])

        def wait_out(b):
            pltpu.make_async_copy(
                touts[b], wf_hbm.at[pl.ds(0, blk)], osems[b]).wait()

        def step(k, b, first, last):
            q = k * _NW + wid
            wait_in(b)
            if not first:
                wait_out(b)
            transpose(b)
            if not last:
                fire_in((k + 2) * _NW + wid, b)
            fire_out(q, b)

        fire_in(wid, 0)
        fire_in(_NW + wid, 1)
        step(0, 0, True, False)
        step(1, 1, True, False)

        def outer(kk, _):
            step(2 * kk, 0, False, False)
            step(2 * kk + 1, 1, False, False)
            return ()

        lax.fori_loop(1, k_main // 2 - 1, outer, (), unroll=False)
        step(k_main - 2, 0, False, True)
        step(k_main - 1, 1, False, True)
        wait_out(0)
        wait_out(1)

        @pl.when(wid < n_extra)
        def _():
            q = n_even + wid
            fire_in(q, 0)
            wait_in(0)
            transpose(0)
            fire_out(q, 0)
            wait_out(0)

        if tail:
            @pl.when(wid == _NW - 1)
            def _():
                pltpu.sync_copy(wtail_hbm, tail_v)

                def shuf(t, _):
                    f = 16 * t
                    tail_r[pl.ds(f, 16)] = tail_v[f // 128,
                                                  pl.ds(f % 128, 16)]
                    return ()

                lax.fori_loop(0, tail * d // 16, shuf, (), unroll=4)
                pltpu.sync_copy(
                    tail_r, wf_hbm.at[pl.ds(n_full * blk, tail * d)])

    return body(wt, wtail)


def _gather(xt, w2):
    seq, n_rows = xt.shape
    vocab_pad, d = w2.shape
    ipw = n_rows // _NW

    mesh = plsc.VectorSubcoreMesh(
        core_axis_name="c", subcore_axis_name="s")

    @functools.partial(
        pl.kernel,
        mesh=mesh,
        compiler_params=pltpu.CompilerParams(
            use_tc_tiling_on_sc=False, needs_layout_passes=False),
        out_type=jax.ShapeDtypeStruct(
            (seq, d // 8, n_rows // ipw, 8, ipw), jnp.float32),
        scratch_types=[
            pltpu.VMEM((seq, ipw), jnp.int32),
            pltpu.VMEM((2, ipw, d), jnp.float32),
            pltpu.VMEM((2, 1, 4, 1, 8, ipw), jnp.float32),
            [pltpu.SemaphoreType.DMA] * 2,
            [pltpu.SemaphoreType.DMA] * 2,
            pltpu.SemaphoreType.DMA,
        ],
    )
    def body(xt_hbm, w2_hbm, o5_hbm, idx_v, grows, oblk,
             gsems, osems, xsem):
        cid = lax.axis_index("c")
        sid = lax.axis_index("s")
        wid = sid * _NC + cid
        i0 = wid * ipw

        lane = lax.iota(jnp.int32, 16)
        row_idx = [lane + 16 * m for m in range(8)]

        pltpu.sync_copy(xt_hbm.at[:, pl.ds(i0, ipw)], idx_v)

        def fire_gather(j, b):
            pltpu.async_copy(
                w2_hbm.at[idx_v.at[j]], grows.at[b], gsems[b])

        def wait_gather(b):
            pltpu.make_async_copy(
                w2_hbm.at[idx_v.at[0]], grows.at[b], gsems[b]).wait()

        def transpose(b):
            def col(c, _):
                cvec = jnp.full((16,), c, jnp.int32)
                for m in range(8):
                    v = plsc.load_gather(grows.at[b], [row_idx[m], cvec])
                    oblk[b, 0, c // 8, 0, c % 8, pl.ds(16 * m, 16)] = v
                return ()

            lax.fori_loop(0, d, col, (), unroll=4)

        def fire_out(j, b):
            pltpu.async_copy(
                oblk.at[b],
                o5_hbm.at[pl.ds(j, 1), :, pl.ds(wid, 1), :, :], osems[b])

        def wait_out(b):
            pltpu.make_async_copy(
                oblk.at[b],
                o5_hbm.at[pl.ds(0, 1), :, pl.ds(0, 1), :, :],
                osems[b]).wait()

        def step(j, b, first, last):
            wait_gather(b)
            if not first:
                wait_out(b)
            transpose(b)
            if not last:
                fire_gather(j + 2, b)
            fire_out(j, b)

        fire_gather(0, 0)
        fire_gather(1, 1)
        step(0, 0, True, False)
        step(1, 1, True, False)

        def outer(jj, _):
            step(2 * jj, 0, False, False)
            step(2 * jj + 1, 1, False, False)
            return ()

        lax.fori_loop(1, seq // 2 - 1, outer, (), unroll=False)
        step(seq - 2, 0, False, True)
        step(seq - 1, 1, False, True)
        wait_out(0)
        wait_out(1)

    return body(xt, w2)


def kernel(x, w):
    vocab, d = w.shape
    n_full = vocab // 128
    tail = vocab - n_full * 128
    vocab_pad = (vocab + 127) // 128 * 128
    wtail = w[n_full * 128:].reshape(tail * d // 128, 128)
    wf = _build_table(w.T, wtail)
    w2 = wf.reshape(vocab_pad, d)
    o5 = _gather(x.T, w2)
    b, s = x.shape
    return o5.transpose(2, 4, 0, 1, 3).reshape(b, s, d)


# speedup vs baseline: 1.1894x; 1.1894x over previous
import functools

import jax
import jax.numpy as jnp
from jax import lax
from jax.experimental import pallas as pl
from jax.experimental.pallas import tpu as pltpu
from jax.experimental.pallas import tpu_sc as plsc

_NC = 2
_NS = 16
_NW = _NC * _NS


def _build_table(wt, wtail):
    d, vocab = wt.shape
    n_full = vocab // 128
    vocab_pad = (vocab + 127) // 128 * 128
    n_even = (n_full // _NW) * _NW
    k_main = n_even // _NW
    n_extra = n_full - n_even
    tail = vocab - n_full * 128
    blk = 128 * d

    mesh = plsc.VectorSubcoreMesh(
        core_axis_name="c", subcore_axis_name="s")

    @functools.partial(
        pl.kernel,
        mesh=mesh,
        compiler_params=pltpu.CompilerParams(
            use_tc_tiling_on_sc=True, needs_layout_passes=False),
        out_type=jax.ShapeDtypeStruct((vocab_pad * d,), jnp.float32),
        scratch_types=[
            pltpu.VMEM((2, 4, 8, 128), jnp.float32),
            [pltpu.VMEM((blk,), jnp.float32)] * 2,
            pltpu.VMEM((tail * d // 128, 128), jnp.float32),
            pltpu.VMEM((tail * d,), jnp.float32),
            [pltpu.SemaphoreType.DMA] * 2,
            [pltpu.SemaphoreType.DMA] * 2,
        ],
    )
    def body(wt_hbm, wtail_hbm, wf_hbm, tin, touts, tail_v, tail_r,
             isems, osems):
        cid = lax.axis_index("c")
        sid = lax.axis_index("s")
        wid = sid * _NC + cid

        lane32 = lax.iota(jnp.int32, 16) * d

        def fire_in(q, b):
            for a in range(4):
                pltpu.async_copy(
                    wt_hbm.at[pl.ds(8 * a, 8), pl.ds(q * 128, 128)],
                    tin.at[b, a], isems[b])

        def wait_in(b):
            for a in range(4):
                pltpu.make_async_copy(
                    wt_hbm.at[pl.ds(0, 8), pl.ds(0, 128)],
                    tin.at[b, a], isems[b]).wait()

        def transpose(b):
            @plsc.parallel_loop(0, d, unroll=4)
            def _(c):
                vs = [tin[b, c // 8, c % 8, pl.ds(16 * m, 16)]
                      for m in range(8)]
                for m in range(8):
                    plsc.store_scatter(
                        touts[b], [lane32 + (16 * m * d + c)], vs[m])

        def fire_out(q, b):
            pltpu.async_copy(
                touts[b], wf_hbm.at[pl.ds(q * blk, blk)], osems[b])

        def wait_out(b):
            pltpu.make_async_copy(
                touts[b], wf_hbm.at[pl.ds(0, blk)], osems[b]).wait()

        def step(k, b, first, last):
            q = k * _NW + wid
            wait_in(b)
            if not first:
                wait_out(b)
            transpose(b)
            if not last:
                fire_in((k + 2) * _NW + wid, b)
            fire_out(q, b)

        fire_in(wid, 0)
        fire_in(_NW + wid, 1)
        step(0, 0, True, False)
        step(1, 1, True, False)

        def outer(kk, _):
            step(2 * kk, 0, False, False)
            step(2 * kk + 1, 1, False, False)
            return ()

        lax.fori_loop(1, k_main // 2 - 1, outer, (), unroll=False)
        step(k_main - 2, 0, False, True)
        step(k_main - 1, 1, False, True)
        wait_out(0)
        wait_out(1)

        @pl.when(wid < n_extra)
        def _():
            q = n_even + wid
            fire_in(q, 0)
            wait_in(0)
            transpose(0)
            fire_out(q, 0)
            wait_out(0)

        if tail:
            @pl.when(wid == _NW - 1)
            def _():
                pltpu.sync_copy(wtail_hbm, tail_v)

                def shuf(t, _):
                    f = 16 * t
                    tail_r[pl.ds(f, 16)] = tail_v[f // 128,
                                                  pl.ds(f % 128, 16)]
                    return ()

                lax.fori_loop(0, tail * d // 16, shuf, (), unroll=4)
                pltpu.sync_copy(
                    tail_r, wf_hbm.at[pl.ds(n_full * blk, tail * d)])

    return body(wt, wtail)


def _gather(xt, w2):
    seq, n_rows = xt.shape
    vocab_pad, d = w2.shape
    ipw = n_rows // _NW

    mesh = plsc.VectorSubcoreMesh(
        core_axis_name="c", subcore_axis_name="s")

    @functools.partial(
        pl.kernel,
        mesh=mesh,
        compiler_params=pltpu.CompilerParams(
            use_tc_tiling_on_sc=False, needs_layout_passes=False),
        out_type=jax.ShapeDtypeStruct(
            (seq, d // 8, n_rows // ipw, 8, ipw), jnp.float32),
        scratch_types=[
            pltpu.VMEM((seq, ipw), jnp.int32),
            pltpu.VMEM((2, ipw, d), jnp.float32),
            pltpu.VMEM((2, 1, 4, 1, 8, ipw), jnp.float32),
            [pltpu.SemaphoreType.DMA] * 2,
            [pltpu.SemaphoreType.DMA] * 2,
            pltpu.SemaphoreType.DMA,
        ],
    )
    def body(xt_hbm, w2_hbm, o5_hbm, idx_v, grows, oblk,
             gsems, osems, xsem):
        cid = lax.axis_index("c")
        sid = lax.axis_index("s")
        wid = sid * _NC + cid
        i0 = wid * ipw

        lane = lax.iota(jnp.int32, 16)
        row_idx = [lane + 16 * m for m in range(8)]

        pltpu.sync_copy(xt_hbm.at[:, pl.ds(i0, ipw)], idx_v)

        def fire_gather(j, b):
            pltpu.async_copy(
                w2_hbm.at[idx_v.at[j]], grows.at[b], gsems[b])

        def wait_gather(b):
            pltpu.make_async_copy(
                w2_hbm.at[idx_v.at[0]], grows.at[b], gsems[b]).wait()

        def transpose(b):
            @plsc.parallel_loop(0, d, unroll=4)
            def _(c):
                cvec = jnp.full((16,), c, jnp.int32)
                vs = [plsc.load_gather(grows.at[b], [row_idx[m], cvec])
                      for m in range(8)]
                for m in range(8):
                    oblk[b, 0, c // 8, 0, c % 8, pl.ds(16 * m, 16)] = vs[m]

        def fire_out(j, b):
            pltpu.async_copy(
                oblk.at[b],
                o5_hbm.at[pl.ds(j, 1), :, pl.ds(wid, 1), :, :], osems[b])

        def wait_out(b):
            pltpu.make_async_copy(
                oblk.at[b],
                o5_hbm.at[pl.ds(0, 1), :, pl.ds(0, 1), :, :],
                osems[b]).wait()

        def step(j, b, first, last):
            wait_gather(b)
            if not first:
                wait_out(b)
            transpose(b)
            if not last:
                fire_gather(j + 2, b)
            fire_out(j, b)

        fire_gather(0, 0)
        fire_gather(1, 1)
        step(0, 0, True, False)
        step(1, 1, True, False)

        def outer(jj, _):
            step(2 * jj, 0, False, False)
            step(2 * jj + 1, 1, False, False)
            return ()

        lax.fori_loop(1, seq // 2 - 1, outer, (), unroll=False)
        step(seq - 2, 0, False, True)
        step(seq - 1, 1, False, True)
        wait_out(0)
        wait_out(1)

    return body(xt, w2)


def kernel(x, w):
    vocab, d = w.shape
    n_full = vocab // 128
    tail = vocab - n_full * 128
    vocab_pad = (vocab + 127) // 128 * 128
    wtail = w[n_full * 128:].reshape(tail * d // 128, 128)
    wf = _build_table(w.T, wtail)
    w2 = wf.reshape(vocab_pad, d)
    o5 = _gather(x.T, w2)
    b, s = x.shape
    return o5.transpose(2, 4, 0, 1, 3).reshape(b, s, d)


# speedup vs baseline: 4.1542x; 3.4928x over previous
import functools

import jax
import jax.numpy as jnp
from jax import lax
from jax.experimental import pallas as pl
from jax.experimental.pallas import tpu as pltpu
from jax.experimental.pallas import tpu_sc as plsc

_NC = 2
_NS = 16
_NW = _NC * _NS


def _build_table(wt, wtail):
    d, vocab = wt.shape
    n_full = vocab // 128
    vocab_pad = (vocab + 127) // 128 * 128
    n_even = (n_full // _NW) * _NW
    k_main = n_even // _NW
    n_extra = n_full - n_even
    tail = vocab - n_full * 128
    blk = 128 * d

    mesh = plsc.VectorSubcoreMesh(
        core_axis_name="c", subcore_axis_name="s")

    @functools.partial(
        pl.kernel,
        mesh=mesh,
        compiler_params=pltpu.CompilerParams(
            use_tc_tiling_on_sc=True, needs_layout_passes=False),
        out_type=jax.ShapeDtypeStruct((vocab_pad * d,), jnp.float32),
        scratch_types=[
            pltpu.VMEM((2, 4, 8, 128), jnp.float32),
            [pltpu.VMEM((blk,), jnp.float32)] * 2,
            pltpu.VMEM((tail * d // 128, 128), jnp.float32),
            pltpu.VMEM((tail * d,), jnp.float32),
            [pltpu.SemaphoreType.DMA] * 2,
            [pltpu.SemaphoreType.DMA] * 2,
        ],
    )
    def body(wt_hbm, wtail_hbm, wf_hbm, tin, touts, tail_v, tail_r,
             isems, osems):
        cid = lax.axis_index("c")
        sid = lax.axis_index("s")
        wid = sid * _NC + cid

        lane = lax.iota(jnp.int32, 16)
        avecs = {c0: (lane + c0) // 8 for c0 in (0, 16)}
        svecs = {c0: (lane + c0) % 8 for c0 in (0, 16)}
        cvecs = {c0: lane + c0 for c0 in (0, 16)}

        def fire_in(q, b):
            for a in range(4):
                pltpu.async_copy(
                    wt_hbm.at[pl.ds(8 * a, 8), pl.ds(q * 128, 128)],
                    tin.at[b, a], isems[b])

        def wait_in(b):
            for a in range(4):
                pltpu.make_async_copy(
                    wt_hbm.at[pl.ds(0, 8), pl.ds(0, 128)],
                    tin.at[b, a], isems[b]).wait()

        def transpose(b):
            @plsc.parallel_loop(0, 128, unroll=4)
            def _(l0):
                lp = (lane + l0) & 127
                lp32 = lp * d
                for c0 in (0, 16):
                    v = plsc.load_gather(
                        tin.at[b], [avecs[c0], svecs[c0], lp])
                    plsc.store_scatter(touts[b], [lp32 + cvecs[c0]], v)

        def fire_out(q, b):
            pltpu.async_copy(
                touts[b], wf_hbm.at[pl.ds(q * blk, blk)], osems[b])

        def wait_out(b):
            pltpu.make_async_copy(
                touts[b], wf_hbm.at[pl.ds(0, blk)], osems[b]).wait()

        def step(k, b, first, last):
            q = k * _NW + wid
            wait_in(b)
            if not first:
                wait_out(b)
            transpose(b)
            if not last:
                fire_in((k + 2) * _NW + wid, b)
            fire_out(q, b)

        fire_in(wid, 0)
        fire_in(_NW + wid, 1)
        step(0, 0, True, False)
        step(1, 1, True, False)

        def outer(kk, _):
            step(2 * kk, 0, False, False)
            step(2 * kk + 1, 1, False, False)
            return ()

        lax.fori_loop(1, k_main // 2 - 1, outer, (), unroll=False)
        step(k_main - 2, 0, False, True)
        step(k_main - 1, 1, False, True)
        wait_out(0)
        wait_out(1)

        @pl.when(wid < n_extra)
        def _():
            q = n_even + wid
            fire_in(q, 0)
            wait_in(0)
            transpose(0)
            fire_out(q, 0)
            wait_out(0)

        if tail:
            @pl.when(wid == _NW - 1)
            def _():
                pltpu.sync_copy(wtail_hbm, tail_v)

                def shuf(t, _):
                    f = 16 * t
                    tail_r[pl.ds(f, 16)] = tail_v[f // 128,
                                                  pl.ds(f % 128, 16)]
                    return ()

                lax.fori_loop(0, tail * d // 16, shuf, (), unroll=4)
                pltpu.sync_copy(
                    tail_r, wf_hbm.at[pl.ds(n_full * blk, tail * d)])

    return body(wt, wtail)


def _gather(xt, w2):
    seq, n_rows = xt.shape
    vocab_pad, d = w2.shape
    ipw = n_rows // _NW

    mesh = plsc.VectorSubcoreMesh(
        core_axis_name="c", subcore_axis_name="s")

    @functools.partial(
        pl.kernel,
        mesh=mesh,
        compiler_params=pltpu.CompilerParams(
            use_tc_tiling_on_sc=False, needs_layout_passes=False),
        out_type=jax.ShapeDtypeStruct(
            (seq, d // 8, n_rows // ipw, 8, ipw), jnp.float32),
        scratch_types=[
            pltpu.VMEM((seq, ipw), jnp.int32),
            pltpu.VMEM((2, ipw, d), jnp.float32),
            pltpu.VMEM((2, 1, 4, 1, 8, ipw), jnp.float32),
            [pltpu.SemaphoreType.DMA] * 2,
            [pltpu.SemaphoreType.DMA] * 2,
            pltpu.SemaphoreType.DMA,
        ],
    )
    def body(xt_hbm, w2_hbm, o5_hbm, idx_v, grows, oblk,
             gsems, osems, xsem):
        cid = lax.axis_index("c")
        sid = lax.axis_index("s")
        wid = sid * _NC + cid
        i0 = wid * ipw

        lane = lax.iota(jnp.int32, 16)
        zvec = jnp.zeros((16,), jnp.int32)
        avecs = {c0: (lane + c0) // 8 for c0 in (0, 16)}
        svecs = {c0: (lane + c0) % 8 for c0 in (0, 16)}
        cvecs = {c0: lane + c0 for c0 in (0, 16)}

        pltpu.sync_copy(xt_hbm.at[:, pl.ds(i0, ipw)], idx_v)

        def fire_gather(j, b):
            pltpu.async_copy(
                w2_hbm.at[idx_v.at[j]], grows.at[b], gsems[b])

        def wait_gather(b):
            pltpu.make_async_copy(
                w2_hbm.at[idx_v.at[0]], grows.at[b], gsems[b]).wait()

        def transpose(b):
            @plsc.parallel_loop(0, ipw, unroll=4)
            def _(l0):
                lp = (lane + l0) & 127
                for c0 in (0, 16):
                    v = plsc.load_gather(grows.at[b], [lp, cvecs[c0]])
                    plsc.store_scatter(
                        oblk.at[b],
                        [zvec, avecs[c0], zvec, svecs[c0], lp], v)

        def fire_out(j, b):
            pltpu.async_copy(
                oblk.at[b],
                o5_hbm.at[pl.ds(j, 1), :, pl.ds(wid, 1), :, :], osems[b])

        def wait_out(b):
            pltpu.make_async_copy(
                oblk.at[b],
                o5_hbm.at[pl.ds(0, 1), :, pl.ds(0, 1), :, :],
                osems[b]).wait()

        def step(j, b, first, last):
            wait_gather(b)
            if not first:
                wait_out(b)
            transpose(b)
            if not last:
                fire_gather(j + 2, b)
            fire_out(j, b)

        fire_gather(0, 0)
        fire_gather(1, 1)
        step(0, 0, True, False)
        step(1, 1, True, False)

        def outer(jj, _):
            step(2 * jj, 0, False, False)
            step(2 * jj + 1, 1, False, False)
            return ()

        lax.fori_loop(1, seq // 2 - 1, outer, (), unroll=False)
        step(seq - 2, 0, False, True)
        step(seq - 1, 1, False, True)
        wait_out(0)
        wait_out(1)

    return body(xt, w2)


def kernel(x, w):
    vocab, d = w.shape
    n_full = vocab // 128
    tail = vocab - n_full * 128
    vocab_pad = (vocab + 127) // 128 * 128
    wtail = w[n_full * 128:].reshape(tail * d // 128, 128)
    wf = _build_table(w.T, wtail)
    w2 = wf.reshape(vocab_pad, d)
    o5 = _gather(x.T, w2)
    b, s = x.shape
    return o5.transpose(2, 4, 0, 1, 3).reshape(b, s, d)


# speedup vs baseline: 4.8802x; 1.1748x over previous
import functools

import jax
import jax.numpy as jnp
from jax import lax
from jax.experimental import pallas as pl
from jax.experimental.pallas import tpu as pltpu
from jax.experimental.pallas import tpu_sc as plsc

_NC = 2
_NS = 16
_NW = _NC * _NS


def _build_table(wt, wtail):
    d, vocab = wt.shape
    n_full = vocab // 128
    vocab_pad = (vocab + 127) // 128 * 128
    n_even = (n_full // _NW) * _NW
    k_main = n_even // _NW
    n_extra = n_full - n_even
    tail = vocab - n_full * 128
    blk = 128 * d

    mesh = plsc.VectorSubcoreMesh(
        core_axis_name="c", subcore_axis_name="s")

    @functools.partial(
        pl.kernel,
        mesh=mesh,
        compiler_params=pltpu.CompilerParams(
            use_tc_tiling_on_sc=True, needs_layout_passes=False),
        out_type=jax.ShapeDtypeStruct((vocab_pad * d,), jnp.float32),
        scratch_types=[
            pltpu.VMEM((2, 4, 8, 128), jnp.float32),
            [pltpu.VMEM((blk,), jnp.float32)] * 2,
            pltpu.VMEM((tail * d // 128, 128), jnp.float32),
            pltpu.VMEM((tail * d,), jnp.float32),
            [pltpu.SemaphoreType.DMA] * 2,
            [pltpu.SemaphoreType.DMA] * 2,
        ],
    )
    def body(wt_hbm, wtail_hbm, wf_hbm, tin, touts, tail_v, tail_r,
             isems, osems):
        cid = lax.axis_index("c")
        sid = lax.axis_index("s")
        wid = sid * _NC + cid

        lane = lax.iota(jnp.int32, 16)
        avecs = {c0: (lane + c0) // 8 for c0 in (0, 16)}
        svecs = {c0: (lane + c0) % 8 for c0 in (0, 16)}
        cvecs = {c0: lane + c0 for c0 in (0, 16)}

        def fire_in(q, b):
            for a in range(4):
                pltpu.async_copy(
                    wt_hbm.at[pl.ds(8 * a, 8), pl.ds(q * 128, 128)],
                    tin.at[b, a], isems[b])

        def wait_in(b):
            for a in range(4):
                pltpu.make_async_copy(
                    wt_hbm.at[pl.ds(0, 8), pl.ds(0, 128)],
                    tin.at[b, a], isems[b]).wait()

        def transpose(b):
            @plsc.parallel_loop(0, 128, unroll=4)
            def _(l0):
                lp = (lane + l0) & 127
                lp32 = lp * d
                for c0 in (0, 16):
                    v = plsc.load_gather(
                        tin.at[b], [avecs[c0], svecs[c0], lp])
                    plsc.store_scatter(touts[b], [lp32 + cvecs[c0]], v)

        def fire_out(q, b):
            pltpu.async_copy(
                touts[b], wf_hbm.at[pl.ds(q * blk, blk)], osems[b])

        def wait_out(b):
            pltpu.make_async_copy(
                touts[b], wf_hbm.at[pl.ds(0, blk)], osems[b]).wait()

        def step(k, b, first, last):
            q = k * _NW + wid
            wait_in(b)
            if not first:
                wait_out(b)
            transpose(b)
            if not last:
                fire_in((k + 2) * _NW + wid, b)
            fire_out(q, b)

        fire_in(wid, 0)
        fire_in(_NW + wid, 1)
        step(0, 0, True, False)
        step(1, 1, True, False)

        def outer(kk, _):
            step(2 * kk, 0, False, False)
            step(2 * kk + 1, 1, False, False)
            return ()

        lax.fori_loop(1, k_main // 2 - 1, outer, (), unroll=False)
        step(k_main - 2, 0, False, True)
        step(k_main - 1, 1, False, True)
        wait_out(0)
        wait_out(1)

        @pl.when(wid < n_extra)
        def _():
            q = n_even + wid
            fire_in(q, 0)
            wait_in(0)
            transpose(0)
            fire_out(q, 0)
            wait_out(0)

        if tail:
            @pl.when(wid == _NW - 1)
            def _():
                pltpu.sync_copy(wtail_hbm, tail_v)

                def shuf(t, _):
                    f = 16 * t
                    tail_r[pl.ds(f, 16)] = tail_v[f // 128,
                                                  pl.ds(f % 128, 16)]
                    return ()

                lax.fori_loop(0, tail * d // 16, shuf, (), unroll=4)
                pltpu.sync_copy(
                    tail_r, wf_hbm.at[pl.ds(n_full * blk, tail * d)])

    return body(wt, wtail)


def _gather(xt, w2):
    seq, n_rows = xt.shape
    vocab_pad, d = w2.shape
    ipw = n_rows // _NW

    mesh = plsc.VectorSubcoreMesh(
        core_axis_name="c", subcore_axis_name="s")

    @functools.partial(
        pl.kernel,
        mesh=mesh,
        compiler_params=pltpu.CompilerParams(
            use_tc_tiling_on_sc=False, needs_layout_passes=False),
        out_type=jax.ShapeDtypeStruct(
            (seq, d // 8, n_rows // ipw, 8, ipw), jnp.float32),
        scratch_types=[
            pltpu.VMEM((seq, ipw), jnp.int32),
            pltpu.VMEM((4, ipw, d), jnp.float32),
            pltpu.VMEM((4, 1, 4, 1, 8, ipw), jnp.float32),
            [pltpu.SemaphoreType.DMA] * 4,
            [pltpu.SemaphoreType.DMA] * 4,
            pltpu.SemaphoreType.DMA,
        ],
    )
    def body(xt_hbm, w2_hbm, o5_hbm, idx_v, grows, oblk,
             gsems, osems, xsem):
        cid = lax.axis_index("c")
        sid = lax.axis_index("s")
        wid = sid * _NC + cid
        i0 = wid * ipw

        lane = lax.iota(jnp.int32, 16)
        zvec = jnp.zeros((16,), jnp.int32)
        avecs = {c0: (lane + c0) // 8 for c0 in (0, 16)}
        svecs = {c0: (lane + c0) % 8 for c0 in (0, 16)}
        cvecs = {c0: lane + c0 for c0 in (0, 16)}

        pltpu.sync_copy(xt_hbm.at[:, pl.ds(i0, ipw)], idx_v)

        def fire_gather(j, b):
            pltpu.async_copy(
                w2_hbm.at[idx_v.at[j]], grows.at[b], gsems[b])

        def wait_gather(b):
            pltpu.make_async_copy(
                w2_hbm.at[idx_v.at[0]], grows.at[b], gsems[b]).wait()

        def transpose(b):
            @plsc.parallel_loop(0, ipw, unroll=4)
            def _(l0):
                lp = (lane + l0) & 127
                for c0 in (0, 16):
                    v = plsc.load_gather(grows.at[b], [lp, cvecs[c0]])
                    plsc.store_scatter(
                        oblk.at[b],
                        [zvec, avecs[c0], zvec, svecs[c0], lp], v)

        def fire_out(j, b):
            pltpu.async_copy(
                oblk.at[b],
                o5_hbm.at[pl.ds(j, 1), :, pl.ds(wid, 1), :, :], osems[b])

        def wait_out(b):
            pltpu.make_async_copy(
                oblk.at[b],
                o5_hbm.at[pl.ds(0, 1), :, pl.ds(0, 1), :, :],
                osems[b]).wait()

        nb = 4

        def step(j, b, first, last):
            wait_gather(b)
            if not first:
                wait_out(b)
            transpose(b)
            if not last:
                fire_gather(j + nb, b)
            fire_out(j, b)

        for b in range(nb):
            fire_gather(b, b)
        for b in range(nb):
            step(b, b, True, False)

        def outer(jj, _):
            for b in range(nb):
                step(nb * jj + b, b, False, False)
            return ()

        lax.fori_loop(1, seq // nb - 1, outer, (), unroll=False)
        for b in range(nb):
            step(seq - nb + b, b, False, True)
        for b in range(nb):
            wait_out(b)

    return body(xt, w2)


def kernel(x, w):
    vocab, d = w.shape
    n_full = vocab // 128
    tail = vocab - n_full * 128
    vocab_pad = (vocab + 127) // 128 * 128
    wtail = w[n_full * 128:].reshape(tail * d // 128, 128)
    wf = _build_table(w.T, wtail)
    w2 = wf.reshape(vocab_pad, d)
    o5 = _gather(x.T, w2)
    b, s = x.shape
    return o5.transpose(2, 4, 0, 1, 3).reshape(b, s, d)


# speedup vs baseline: 6.2283x; 1.2762x over previous
import functools

import jax
import jax.numpy as jnp
from jax import lax
from jax.experimental import pallas as pl
from jax.experimental.pallas import tpu as pltpu
from jax.experimental.pallas import tpu_sc as plsc

_NC = 2
_NS = 16
_NW = _NC * _NS


def _build_table(wt, wtail):
    d, vocab = wt.shape
    n_full = vocab // 128
    vocab_pad = (vocab + 127) // 128 * 128
    n_even = (n_full // _NW) * _NW
    k_main = n_even // _NW
    n_extra = n_full - n_even
    tail = vocab - n_full * 128
    blk = 128 * d

    mesh = plsc.VectorSubcoreMesh(
        core_axis_name="c", subcore_axis_name="s")

    @functools.partial(
        pl.kernel,
        mesh=mesh,
        compiler_params=pltpu.CompilerParams(
            use_tc_tiling_on_sc=True, needs_layout_passes=False),
        out_type=jax.ShapeDtypeStruct((vocab_pad * d,), jnp.float32),
        scratch_types=[
            pltpu.VMEM((4, 4, 8, 128), jnp.float32),
            [pltpu.VMEM((blk,), jnp.float32)] * 4,
            pltpu.VMEM((tail * d // 128, 128), jnp.float32),
            pltpu.VMEM((tail * d,), jnp.float32),
            [pltpu.SemaphoreType.DMA] * 4,
            [pltpu.SemaphoreType.DMA] * 4,
        ],
    )
    def body(wt_hbm, wtail_hbm, wf_hbm, tin, touts, tail_v, tail_r,
             isems, osems):
        cid = lax.axis_index("c")
        sid = lax.axis_index("s")
        wid = sid * _NC + cid

        lane = lax.iota(jnp.int32, 16)
        avecs = {c0: (lane + c0) // 8 for c0 in (0, 16)}
        svecs = {c0: (lane + c0) % 8 for c0 in (0, 16)}
        cvecs = {c0: lane + c0 for c0 in (0, 16)}

        def fire_in(q, b):
            for a in range(4):
                pltpu.async_copy(
                    wt_hbm.at[pl.ds(8 * a, 8), pl.ds(q * 128, 128)],
                    tin.at[b, a], isems[b])

        def wait_in(b):
            for a in range(4):
                pltpu.make_async_copy(
                    wt_hbm.at[pl.ds(0, 8), pl.ds(0, 128)],
                    tin.at[b, a], isems[b]).wait()

        def transpose(b):
            @plsc.parallel_loop(0, 128, unroll=4)
            def _(l0):
                lp = (lane + l0) & 127
                lp32 = lp * d
                for c0 in (0, 16):
                    v = plsc.load_gather(
                        tin.at[b], [avecs[c0], svecs[c0], lp])
                    plsc.store_scatter(touts[b], [lp32 + cvecs[c0]], v)

        def fire_out(q, b):
            pltpu.async_copy(
                touts[b], wf_hbm.at[pl.ds(q * blk, blk)], osems[b])

        def wait_out(b):
            pltpu.make_async_copy(
                touts[b], wf_hbm.at[pl.ds(0, blk)], osems[b]).wait()

        nb = 4

        def step(k, b, first, last):
            q = k * _NW + wid
            wait_in(b)
            if not first:
                wait_out(b)
            transpose(b)
            if not last:
                fire_in((k + nb) * _NW + wid, b)
            fire_out(q, b)

        for b in range(nb):
            fire_in(b * _NW + wid, b)
        for b in range(nb):
            step(b, b, True, False)

        def outer(kk, _):
            for b in range(nb):
                step(nb * kk + b, b, False, False)
            return ()

        lax.fori_loop(1, k_main // nb - 1, outer, (), unroll=False)
        for b in range(nb):
            step(k_main - nb + b, b, False, True)
        for b in range(nb):
            wait_out(b)

        @pl.when(wid < n_extra)
        def _():
            q = n_even + wid
            fire_in(q, 0)
            wait_in(0)
            transpose(0)
            fire_out(q, 0)
            wait_out(0)

        if tail:
            @pl.when(wid == _NW - 1)
            def _():
                pltpu.sync_copy(wtail_hbm, tail_v)

                def shuf(t, _):
                    f = 16 * t
                    tail_r[pl.ds(f, 16)] = tail_v[f // 128,
                                                  pl.ds(f % 128, 16)]
                    return ()

                lax.fori_loop(0, tail * d // 16, shuf, (), unroll=4)
                pltpu.sync_copy(
                    tail_r, wf_hbm.at[pl.ds(n_full * blk, tail * d)])

    return body(wt, wtail)


def _gather(xt, w2):
    seq, n_rows = xt.shape
    vocab_pad, d = w2.shape
    ipw = n_rows // _NW

    mesh = plsc.VectorSubcoreMesh(
        core_axis_name="c", subcore_axis_name="s")

    @functools.partial(
        pl.kernel,
        mesh=mesh,
        compiler_params=pltpu.CompilerParams(
            use_tc_tiling_on_sc=False, needs_layout_passes=False),
        out_type=jax.ShapeDtypeStruct(
            (seq, d // 8, n_rows // ipw, 8, ipw), jnp.float32),
        scratch_types=[
            pltpu.VMEM((seq, ipw), jnp.int32),
            pltpu.VMEM((4, ipw, d), jnp.float32),
            pltpu.VMEM((4, 1, 4, 1, 8, ipw), jnp.float32),
            [pltpu.SemaphoreType.DMA] * 4,
            [pltpu.SemaphoreType.DMA] * 4,
            pltpu.SemaphoreType.DMA,
        ],
    )
    def body(xt_hbm, w2_hbm, o5_hbm, idx_v, grows, oblk,
             gsems, osems, xsem):
        cid = lax.axis_index("c")
        sid = lax.axis_index("s")
        wid = sid * _NC + cid
        i0 = wid * ipw

        lane = lax.iota(jnp.int32, 16)
        zvec = jnp.zeros((16,), jnp.int32)
        avecs = {c0: (lane + c0) // 8 for c0 in (0, 16)}
        svecs = {c0: (lane + c0) % 8 for c0 in (0, 16)}
        cvecs = {c0: lane + c0 for c0 in (0, 16)}

        pltpu.sync_copy(xt_hbm.at[:, pl.ds(i0, ipw)], idx_v)

        def fire_gather(j, b):
            pltpu.async_copy(
                w2_hbm.at[idx_v.at[j]], grows.at[b], gsems[b])

        def wait_gather(b):
            pltpu.make_async_copy(
                w2_hbm.at[idx_v.at[0]], grows.at[b], gsems[b]).wait()

        def transpose(b):
            @plsc.parallel_loop(0, ipw, unroll=4)
            def _(l0):
                lp = (lane + l0) & 127
                for c0 in (0, 16):
                    v = plsc.load_gather(grows.at[b], [lp, cvecs[c0]])
                    plsc.store_scatter(
                        oblk.at[b],
                        [zvec, avecs[c0], zvec, svecs[c0], lp], v)

        def fire_out(j, b):
            pltpu.async_copy(
                oblk.at[b],
                o5_hbm.at[pl.ds(j, 1), :, pl.ds(wid, 1), :, :], osems[b])

        def wait_out(b):
            pltpu.make_async_copy(
                oblk.at[b],
                o5_hbm.at[pl.ds(0, 1), :, pl.ds(0, 1), :, :],
                osems[b]).wait()

        nb = 4

        def step(j, b, first, last):
            wait_gather(b)
            if not first:
                wait_out(b)
            transpose(b)
            if not last:
                fire_gather(j + nb, b)
            fire_out(j, b)

        for b in range(nb):
            fire_gather(b, b)
        for b in range(nb):
            step(b, b, True, False)

        def outer(jj, _):
            for b in range(nb):
                step(nb * jj + b, b, False, False)
            return ()

        lax.fori_loop(1, seq // nb - 1, outer, (), unroll=False)
        for b in range(nb):
            step(seq - nb + b, b, False, True)
        for b in range(nb):
            wait_out(b)

    return body(xt, w2)


def kernel(x, w):
    vocab, d = w.shape
    n_full = vocab // 128
    tail = vocab - n_full * 128
    vocab_pad = (vocab + 127) // 128 * 128
    wtail = w[n_full * 128:].reshape(tail * d // 128, 128)
    wf = _build_table(w.T, wtail)
    w2 = wf.reshape(vocab_pad, d)
    o5 = _gather(x.T, w2)
    b, s = x.shape
    return o5.transpose(2, 4, 0, 1, 3).reshape(b, s, d)


# speedup vs baseline: 6.3956x; 1.0269x over previous
import functools

import jax
import jax.numpy as jnp
from jax import lax
from jax.experimental import pallas as pl
from jax.experimental.pallas import tpu as pltpu
from jax.experimental.pallas import tpu_sc as plsc

_NC = 2
_NS = 16
_NW = _NC * _NS


def _build_table(wt, wtail):
    d, vocab = wt.shape
    n_full = vocab // 128
    vocab_pad = (vocab + 127) // 128 * 128
    n_even = (n_full // _NW) * _NW
    k_main = n_even // _NW
    n_extra = n_full - n_even
    tail = vocab - n_full * 128
    blk = 128 * d

    mesh = plsc.VectorSubcoreMesh(
        core_axis_name="c", subcore_axis_name="s")

    @functools.partial(
        pl.kernel,
        mesh=mesh,
        compiler_params=pltpu.CompilerParams(
            use_tc_tiling_on_sc=True, needs_layout_passes=False),
        out_type=jax.ShapeDtypeStruct((vocab_pad * d,), jnp.float32),
        scratch_types=[
            pltpu.VMEM((4, 4, 8, 128), jnp.float32),
            [pltpu.VMEM((blk,), jnp.float32)] * 4,
            pltpu.VMEM((tail * d // 128, 128), jnp.float32),
            pltpu.VMEM((tail * d,), jnp.float32),
            [pltpu.SemaphoreType.DMA] * 4,
            [pltpu.SemaphoreType.DMA] * 4,
        ],
    )
    def body(wt_hbm, wtail_hbm, wf_hbm, tin, touts, tail_v, tail_r,
             isems, osems):
        cid = lax.axis_index("c")
        sid = lax.axis_index("s")
        wid = sid * _NC + cid

        lane = lax.iota(jnp.int32, 16)
        avecs = {c0: (lane + c0) // 8 for c0 in (0, 16)}
        svecs = {c0: (lane + c0) % 8 for c0 in (0, 16)}
        cvecs = {c0: lane + c0 for c0 in (0, 16)}

        def fire_in(q, b):
            for a in range(4):
                pltpu.async_copy(
                    wt_hbm.at[pl.ds(8 * a, 8), pl.ds(q * 128, 128)],
                    tin.at[b, a], isems[b])

        def wait_in(b):
            for a in range(4):
                pltpu.make_async_copy(
                    wt_hbm.at[pl.ds(0, 8), pl.ds(0, 128)],
                    tin.at[b, a], isems[b]).wait()

        def transpose(b):
            @plsc.parallel_loop(0, 128, unroll=4)
            def _(l0):
                lp = (lane + l0) & 127
                lp32 = lp * d
                for c0 in (0, 16):
                    v = plsc.load_gather(
                        tin.at[b], [avecs[c0], svecs[c0], lp])
                    plsc.store_scatter(touts[b], [lp32 + cvecs[c0]], v)

        def fire_out(q, b):
            pltpu.async_copy(
                touts[b], wf_hbm.at[pl.ds(q * blk, blk)], osems[b])

        def wait_out(b):
            pltpu.make_async_copy(
                touts[b], wf_hbm.at[pl.ds(0, blk)], osems[b]).wait()

        nb = 4

        def step(k, b, first, last):
            q = k * _NW + wid
            wait_in(b)
            if not first:
                wait_out(b)
            transpose(b)
            if not last:
                fire_in((k + nb) * _NW + wid, b)
            fire_out(q, b)

        for b in range(nb):
            fire_in(b * _NW + wid, b)
        for b in range(nb):
            step(b, b, True, False)

        def outer(kk, _):
            for b in range(nb):
                step(nb * kk + b, b, False, False)
            return ()

        lax.fori_loop(1, k_main // nb - 1, outer, (), unroll=False)
        for b in range(nb):
            step(k_main - nb + b, b, False, True)
        for b in range(nb):
            wait_out(b)

        @pl.when(wid < n_extra)
        def _():
            q = n_even + wid
            fire_in(q, 0)
            wait_in(0)
            transpose(0)
            fire_out(q, 0)
            wait_out(0)

        if tail:
            @pl.when(wid == _NW - 1)
            def _():
                pltpu.sync_copy(wtail_hbm, tail_v)

                def shuf(t, _):
                    f = 16 * t
                    tail_r[pl.ds(f, 16)] = tail_v[f // 128,
                                                  pl.ds(f % 128, 16)]
                    return ()

                lax.fori_loop(0, tail * d // 16, shuf, (), unroll=4)
                pltpu.sync_copy(
                    tail_r, wf_hbm.at[pl.ds(n_full * blk, tail * d)])

    return body(wt, wtail)


def _gather(xt, w2):
    seq, n_rows = xt.shape
    vocab_pad, d = w2.shape
    ipw = n_rows // _NW

    mesh = plsc.VectorSubcoreMesh(
        core_axis_name="c", subcore_axis_name="s")

    @functools.partial(
        pl.kernel,
        mesh=mesh,
        compiler_params=pltpu.CompilerParams(
            use_tc_tiling_on_sc=False, needs_layout_passes=False),
        out_type=jax.ShapeDtypeStruct(
            (seq, d // 8, n_rows // ipw, 8, ipw), jnp.float32),
        scratch_types=[
            pltpu.VMEM((seq, ipw), jnp.int32),
            pltpu.VMEM((8, ipw, d), jnp.float32),
            pltpu.VMEM((8, 1, 4, 1, 8, ipw), jnp.float32),
            [pltpu.SemaphoreType.DMA] * 8,
            [pltpu.SemaphoreType.DMA] * 8,
            pltpu.SemaphoreType.DMA,
        ],
    )
    def body(xt_hbm, w2_hbm, o5_hbm, idx_v, grows, oblk,
             gsems, osems, xsem):
        cid = lax.axis_index("c")
        sid = lax.axis_index("s")
        wid = sid * _NC + cid
        i0 = wid * ipw

        lane = lax.iota(jnp.int32, 16)
        zvec = jnp.zeros((16,), jnp.int32)
        avecs = {c0: (lane + c0) // 8 for c0 in (0, 16)}
        svecs = {c0: (lane + c0) % 8 for c0 in (0, 16)}
        cvecs = {c0: lane + c0 for c0 in (0, 16)}

        pltpu.sync_copy(xt_hbm.at[:, pl.ds(i0, ipw)], idx_v)

        def fire_gather(j, b):
            pltpu.async_copy(
                w2_hbm.at[idx_v.at[j]], grows.at[b], gsems[b])

        def wait_gather(b):
            pltpu.make_async_copy(
                w2_hbm.at[idx_v.at[0]], grows.at[b], gsems[b]).wait()

        def transpose(b):
            @plsc.parallel_loop(0, ipw, unroll=4)
            def _(l0):
                lp = (lane + l0) & 127
                for c0 in (0, 16):
                    v = plsc.load_gather(grows.at[b], [lp, cvecs[c0]])
                    plsc.store_scatter(
                        oblk.at[b],
                        [zvec, avecs[c0], zvec, svecs[c0], lp], v)

        def fire_out(j, b):
            pltpu.async_copy(
                oblk.at[b],
                o5_hbm.at[pl.ds(j, 1), :, pl.ds(wid, 1), :, :], osems[b])

        def wait_out(b):
            pltpu.make_async_copy(
                oblk.at[b],
                o5_hbm.at[pl.ds(0, 1), :, pl.ds(0, 1), :, :],
                osems[b]).wait()

        nb = 8

        def step(j, b, first, last):
            wait_gather(b)
            if not first:
                wait_out(b)
            transpose(b)
            if not last:
                fire_gather(j + nb, b)
            fire_out(j, b)

        for b in range(nb):
            fire_gather(b, b)
        for b in range(nb):
            step(b, b, True, False)

        def outer(jj, _):
            for b in range(nb):
                step(nb * jj + b, b, False, False)
            return ()

        lax.fori_loop(1, seq // nb - 1, outer, (), unroll=False)
        for b in range(nb):
            step(seq - nb + b, b, False, True)
        for b in range(nb):
            wait_out(b)

    return body(xt, w2)


def kernel(x, w):
    vocab, d = w.shape
    n_full = vocab // 128
    tail = vocab - n_full * 128
    vocab_pad = (vocab + 127) // 128 * 128
    wtail = w[n_full * 128:].reshape(tail * d // 128, 128)
    wf = _build_table(w.T, wtail)
    w2 = wf.reshape(vocab_pad, d)
    o5 = _gather(x.T, w2)
    b, s = x.shape
    return o5.transpose(2, 4, 0, 1, 3).reshape(b, s, d)


# speedup vs baseline: 6.4413x; 1.0071x over previous
import functools

import jax
import jax.numpy as jnp
from jax import lax
from jax.experimental import pallas as pl
from jax.experimental.pallas import tpu as pltpu
from jax.experimental.pallas import tpu_sc as plsc

_NC = 2
_NS = 16
_NW = _NC * _NS


def _build_table(wt, wtail):
    d, vocab = wt.shape
    n_full = vocab // 128
    vocab_pad = (vocab + 127) // 128 * 128
    n_even = (n_full // _NW) * _NW
    k_main = n_even // _NW
    n_extra = n_full - n_even
    tail = vocab - n_full * 128
    blk = 128 * d

    mesh = plsc.VectorSubcoreMesh(
        core_axis_name="c", subcore_axis_name="s")

    @functools.partial(
        pl.kernel,
        mesh=mesh,
        compiler_params=pltpu.CompilerParams(
            use_tc_tiling_on_sc=True, needs_layout_passes=False),
        out_type=jax.ShapeDtypeStruct((vocab_pad * d,), jnp.float32),
        scratch_types=[
            pltpu.VMEM((4, 32, 128), jnp.float32),
            [pltpu.VMEM((blk,), jnp.float32)] * 4,
            pltpu.VMEM((tail * d // 128, 128), jnp.float32),
            pltpu.VMEM((tail * d,), jnp.float32),
            [pltpu.SemaphoreType.DMA] * 4,
            [pltpu.SemaphoreType.DMA] * 4,
        ],
    )
    def body(wt_hbm, wtail_hbm, wf_hbm, tin, touts, tail_v, tail_r,
             isems, osems):
        cid = lax.axis_index("c")
        sid = lax.axis_index("s")
        wid = sid * _NC + cid

        lane = lax.iota(jnp.int32, 16)
        avecs = {c0: (lane + c0) // 8 for c0 in (0, 16)}
        svecs = {c0: (lane + c0) % 8 for c0 in (0, 16)}
        cvecs = {c0: lane + c0 for c0 in (0, 16)}

        def fire_in(q, b):
            pltpu.async_copy(
                wt_hbm.at[:, pl.ds(q * 128, 128)], tin.at[b], isems[b])

        def wait_in(b):
            pltpu.make_async_copy(
                wt_hbm.at[:, pl.ds(0, 128)], tin.at[b], isems[b]).wait()

        def transpose(b):
            @plsc.parallel_loop(0, 128, unroll=4)
            def _(l0):
                lp = (lane + l0) & 127
                lp32 = lp * d
                for c0 in (0, 16):
                    v = plsc.load_gather(tin.at[b], [cvecs[c0], lp])
                    plsc.store_scatter(touts[b], [lp32 + cvecs[c0]], v)

        def fire_out(q, b):
            pltpu.async_copy(
                touts[b], wf_hbm.at[pl.ds(q * blk, blk)], osems[b])

        def wait_out(b):
            pltpu.make_async_copy(
                touts[b], wf_hbm.at[pl.ds(0, blk)], osems[b]).wait()

        nb = 4

        def step(k, b, first, last):
            q = k * _NW + wid
            wait_in(b)
            if not first:
                wait_out(b)
            transpose(b)
            if not last:
                fire_in((k + nb) * _NW + wid, b)
            fire_out(q, b)

        for b in range(nb):
            fire_in(b * _NW + wid, b)
        for b in range(nb):
            step(b, b, True, False)

        def outer(kk, _):
            for b in range(nb):
                step(nb * kk + b, b, False, False)
            return ()

        lax.fori_loop(1, k_main // nb - 1, outer, (), unroll=False)
        for b in range(nb):
            step(k_main - nb + b, b, False, True)
        for b in range(nb):
            wait_out(b)

        @pl.when(wid < n_extra)
        def _():
            q = n_even + wid
            fire_in(q, 0)
            wait_in(0)
            transpose(0)
            fire_out(q, 0)
            wait_out(0)

        if tail:
            @pl.when(wid == _NW - 1)
            def _():
                pltpu.sync_copy(wtail_hbm, tail_v)

                def shuf(t, _):
                    f = 16 * t
                    tail_r[pl.ds(f, 16)] = tail_v[f // 128,
                                                  pl.ds(f % 128, 16)]
                    return ()

                lax.fori_loop(0, tail * d // 16, shuf, (), unroll=4)
                pltpu.sync_copy(
                    tail_r, wf_hbm.at[pl.ds(n_full * blk, tail * d)])

    return body(wt, wtail)


def _gather(xt, w2):
    seq, n_rows = xt.shape
    vocab_pad, d = w2.shape
    ipw = n_rows // _NW

    mesh = plsc.VectorSubcoreMesh(
        core_axis_name="c", subcore_axis_name="s")

    @functools.partial(
        pl.kernel,
        mesh=mesh,
        compiler_params=pltpu.CompilerParams(
            use_tc_tiling_on_sc=False, needs_layout_passes=False),
        out_type=jax.ShapeDtypeStruct(
            (seq, d // 8, n_rows // ipw, 8, ipw), jnp.float32),
        scratch_types=[
            pltpu.VMEM((seq, ipw), jnp.int32),
            pltpu.VMEM((8, ipw, d), jnp.float32),
            pltpu.VMEM((8, 1, 4, 1, 8, ipw), jnp.float32),
            [pltpu.SemaphoreType.DMA] * 8,
            [pltpu.SemaphoreType.DMA] * 8,
            pltpu.SemaphoreType.DMA,
        ],
    )
    def body(xt_hbm, w2_hbm, o5_hbm, idx_v, grows, oblk,
             gsems, osems, xsem):
        cid = lax.axis_index("c")
        sid = lax.axis_index("s")
        wid = sid * _NC + cid
        i0 = wid * ipw

        lane = lax.iota(jnp.int32, 16)
        zvec = jnp.zeros((16,), jnp.int32)
        avecs = {c0: (lane + c0) // 8 for c0 in (0, 16)}
        svecs = {c0: (lane + c0) % 8 for c0 in (0, 16)}
        cvecs = {c0: lane + c0 for c0 in (0, 16)}

        pltpu.sync_copy(xt_hbm.at[:, pl.ds(i0, ipw)], idx_v)

        def fire_gather(j, b):
            pltpu.async_copy(
                w2_hbm.at[idx_v.at[j]], grows.at[b], gsems[b])

        def wait_gather(b):
            pltpu.make_async_copy(
                w2_hbm.at[idx_v.at[0]], grows.at[b], gsems[b]).wait()

        def transpose(b):
            @plsc.parallel_loop(0, ipw, unroll=4)
            def _(l0):
                lp = (lane + l0) & 127
                for c0 in (0, 16):
                    v = plsc.load_gather(grows.at[b], [lp, cvecs[c0]])
                    plsc.store_scatter(
                        oblk.at[b],
                        [zvec, avecs[c0], zvec, svecs[c0], lp], v)

        def fire_out(j, b):
            pltpu.async_copy(
                oblk.at[b],
                o5_hbm.at[pl.ds(j, 1), :, pl.ds(wid, 1), :, :], osems[b])

        def wait_out(b):
            pltpu.make_async_copy(
                oblk.at[b],
                o5_hbm.at[pl.ds(0, 1), :, pl.ds(0, 1), :, :],
                osems[b]).wait()

        nb = 8

        def step(j, b, first, last):
            wait_gather(b)
            if not first:
                wait_out(b)
            transpose(b)
            if not last:
                fire_gather(j + nb, b)
            fire_out(j, b)

        for b in range(nb):
            fire_gather(b, b)
        for b in range(nb):
            step(b, b, True, False)

        def outer(jj, _):
            for b in range(nb):
                step(nb * jj + b, b, False, False)
            return ()

        lax.fori_loop(1, seq // nb - 1, outer, (), unroll=False)
        for b in range(nb):
            step(seq - nb + b, b, False, True)
        for b in range(nb):
            wait_out(b)

    return body(xt, w2)


def kernel(x, w):
    vocab, d = w.shape
    n_full = vocab // 128
    tail = vocab - n_full * 128
    vocab_pad = (vocab + 127) // 128 * 128
    wtail = w[n_full * 128:].reshape(tail * d // 128, 128)
    wf = _build_table(w.T, wtail)
    w2 = wf.reshape(vocab_pad, d)
    o5 = _gather(x.T, w2)
    b, s = x.shape
    return o5.transpose(2, 4, 0, 1, 3).reshape(b, s, d)
